# Initial kernel scaffold; baseline (speedup 1.0000x reference)
import jax
import jax.numpy as jnp
from jax.experimental import pallas as pl


def kernel(x_atom, atom_idx, x_group, group_idx, edge_index_group, cond_atom, g_proj_W, g_proj_b, fg1_W, fg1_b, fg2_W, fg2_b, fb1_W, fb1_b, fb2_W, fb2_b, a2g_W, a2g_b, s2sA_Wih, s2sA_Whh, s2sA_bih, s2sA_bhh, merge_W, merge_b, gcn_msg_W, gcn_msg_b, gcn_self_W, gcn_self_b, gcn_att, s2sG_Wih, s2sG_Whh, s2sG_bih, s2sG_bhh, g2a_W, g2a_b):
    raise NotImplementedError("write your pallas kernel here")


# speedup vs baseline: 1.0739x; 1.0739x over previous
import functools
import jax
import jax.numpy as jnp
from jax.experimental import pallas as pl


def _pick_bm(m):
    for c in (2000, 1000, 500, 250, 125, 100, 50, 25, 8, 5, 4, 2, 1):
        if m % c == 0:
            return c
    return 1


def _mm_body(x_ref, w_ref, b_ref, o_ref, *, act):
    y = jnp.dot(x_ref[...], w_ref[...], preferred_element_type=jnp.float32)
    y = y + b_ref[...]
    if act == "relu":
        y = jnp.maximum(y, 0.0)
    o_ref[...] = y


def _mm(x, W, b, act=None):
    M, K = x.shape
    N = W.shape[0]
    bm = _pick_bm(M)
    Wt = W.T
    b2 = jnp.asarray(b, jnp.float32).reshape(1, N)
    return pl.pallas_call(
        functools.partial(_mm_body, act=act),
        grid=(M // bm,),
        in_specs=[
            pl.BlockSpec((bm, K), lambda i: (i, 0)),
            pl.BlockSpec((K, N), lambda i: (0, 0)),
            pl.BlockSpec((1, N), lambda i: (0, 0)),
        ],
        out_specs=pl.BlockSpec((bm, N), lambda i: (i, 0)),
        out_shape=jax.ShapeDtypeStruct((M, N), jnp.float32),
    )(x, Wt, b2)


def _mm_add_body(x_ref, w_ref, b_ref, a_ref, o_ref):
    y = jnp.dot(x_ref[...], w_ref[...], preferred_element_type=jnp.float32)
    o_ref[...] = y + b_ref[...] + a_ref[...]


def _mm_add(x, W, b, add):
    M, K = x.shape
    N = W.shape[0]
    bm = _pick_bm(M)
    Wt = W.T
    b2 = jnp.asarray(b, jnp.float32).reshape(1, N)
    return pl.pallas_call(
        _mm_add_body,
        grid=(M // bm,),
        in_specs=[
            pl.BlockSpec((bm, K), lambda i: (i, 0)),
            pl.BlockSpec((K, N), lambda i: (0, 0)),
            pl.BlockSpec((1, N), lambda i: (0, 0)),
            pl.BlockSpec((bm, N), lambda i: (i, 0)),
        ],
        out_specs=pl.BlockSpec((bm, N), lambda i: (i, 0)),
        out_shape=jax.ShapeDtypeStruct((M, N), jnp.float32),
    )(x, Wt, b2, add)


def _lstm2_body(g_ref, gc_ref, c1_ref, h_ref, c_ref, *, d):
    g = g_ref[...] + gc_ref[...]
    i = g[:, 0:d]
    f = g[:, d:2 * d]
    gg = g[:, 2 * d:3 * d]
    o = g[:, 3 * d:4 * d]
    c = jax.nn.sigmoid(f) * c1_ref[...] + jax.nn.sigmoid(i) * jnp.tanh(gg)
    h = jax.nn.sigmoid(o) * jnp.tanh(c)
    h_ref[...] = h
    c_ref[...] = c


def _lstm_step2(r, Wih_r, gconst, c1):
    M, d = r.shape
    G = _mm(r, Wih_r, gconst)
    bm = _pick_bm(M)
    c1b = c1.reshape(1, d)
    zero4 = jnp.zeros((1, 4 * d), jnp.float32)
    h, c = pl.pallas_call(
        functools.partial(_lstm2_body, d=d),
        grid=(M // bm,),
        in_specs=[
            pl.BlockSpec((bm, 4 * d), lambda i: (i, 0)),
            pl.BlockSpec((1, 4 * d), lambda i: (0, 0)),
            pl.BlockSpec((1, d), lambda i: (0, 0)),
        ],
        out_specs=[
            pl.BlockSpec((bm, d), lambda i: (i, 0)),
            pl.BlockSpec((bm, d), lambda i: (i, 0)),
        ],
        out_shape=[
            jax.ShapeDtypeStruct((M, d), jnp.float32),
            jax.ShapeDtypeStruct((M, d), jnp.float32),
        ],
    )(G, zero4, c1b)
    return h, c


def _rowdot_body(x_ref, y_ref, o_ref):
    o_ref[...] = jnp.sum(x_ref[...] * y_ref[...], axis=1, keepdims=True)


def _rowdot(x, y):
    M, D = x.shape
    bm = _pick_bm(M)
    return pl.pallas_call(
        _rowdot_body,
        grid=(M // bm,),
        in_specs=[
            pl.BlockSpec((bm, D), lambda i: (i, 0)),
            pl.BlockSpec((bm, D), lambda i: (i, 0)),
        ],
        out_specs=pl.BlockSpec((bm, 1), lambda i: (i, 0)),
        out_shape=jax.ShapeDtypeStruct((M, 1), jnp.float32),
    )(x, y)


def _film_body(cg_ref, w1g_ref, b1g_ref, w2g_ref, b2g_ref,
               w1b_ref, b1b_ref, w2b_ref, b2b_ref, xfa_ref, o_ref):
    cg = cg_ref[...]
    hg = jnp.maximum(jnp.dot(cg, w1g_ref[...],
                             preferred_element_type=jnp.float32)
                     + b1g_ref[...], 0.0)
    gamma = jnp.dot(hg, w2g_ref[...],
                    preferred_element_type=jnp.float32) + b2g_ref[...]
    hb = jnp.maximum(jnp.dot(cg, w1b_ref[...],
                             preferred_element_type=jnp.float32)
                     + b1b_ref[...], 0.0)
    beta = jnp.dot(hb, w2b_ref[...],
                   preferred_element_type=jnp.float32) + b2b_ref[...]
    o_ref[...] = gamma * xfa_ref[...] + beta


def _film(cond_g, fg1_W, fg1_b, fg2_W, fg2_b, fb1_W, fb1_b, fb2_W, fb2_b,
          xg_from_atom):
    M, K = cond_g.shape
    H = fg1_W.shape[0]
    N = fg2_W.shape[0]
    bm = _pick_bm(M)
    args = (cond_g, fg1_W.T, fg1_b.reshape(1, H), fg2_W.T, fg2_b.reshape(1, N),
            fb1_W.T, fb1_b.reshape(1, H), fb2_W.T, fb2_b.reshape(1, N),
            xg_from_atom)
    return pl.pallas_call(
        _film_body,
        grid=(M // bm,),
        in_specs=[
            pl.BlockSpec((bm, K), lambda i: (i, 0)),
            pl.BlockSpec((K, H), lambda i: (0, 0)),
            pl.BlockSpec((1, H), lambda i: (0, 0)),
            pl.BlockSpec((H, N), lambda i: (0, 0)),
            pl.BlockSpec((1, N), lambda i: (0, 0)),
            pl.BlockSpec((K, H), lambda i: (0, 0)),
            pl.BlockSpec((1, H), lambda i: (0, 0)),
            pl.BlockSpec((H, N), lambda i: (0, 0)),
            pl.BlockSpec((1, N), lambda i: (0, 0)),
            pl.BlockSpec((bm, N), lambda i: (i, 0)),
        ],
        out_specs=pl.BlockSpec((bm, N), lambda i: (i, 0)),
        out_shape=jax.ShapeDtypeStruct((M, N), jnp.float32),
    )(*args)


def _lstm_const_step1(Wih, Whh, bih, bhh, d):
    g = bih + bhh
    i, f, gg, o = g[0:d], g[d:2 * d], g[2 * d:3 * d], g[3 * d:4 * d]
    c1 = jax.nn.sigmoid(i) * jnp.tanh(gg)
    h1 = jax.nn.sigmoid(o) * jnp.tanh(c1)
    return h1, c1


def _lstm_step2_consts(Wih, Whh, bih, bhh, h1, d):
    gconst = h1 @ Wih[:, 0:d].T + h1 @ Whh.T + bih + bhh
    Wih_r = Wih[:, d:2 * d]
    return Wih_r, gconst


def _seg_softmax(e, seg, n):
    m = jax.ops.segment_max(e, seg, num_segments=n)
    m = jnp.where(jnp.isfinite(m), m, 0.0)
    ex = jnp.exp(e - m[seg])
    s = jax.ops.segment_sum(ex, seg, num_segments=n)
    return ex / (s[seg] + 1e-16)


def _set2set(x, batch, size, Wih, Whh, bih, bhh, cnt):
    d = x.shape[1]
    h1, c1 = _lstm_const_step1(Wih, Whh, bih, bhh, d)
    e1 = _mm(x, h1.reshape(1, d), jnp.zeros((1,), jnp.float32))[:, 0]
    a1 = _seg_softmax(e1, batch, size)
    r1 = jax.ops.segment_sum(a1[:, None] * x, batch, num_segments=size)
    Wih_r, gconst = _lstm_step2_consts(Wih, Whh, bih, bhh, h1, d)
    h2, c2 = _lstm_step2(r1, Wih_r, gconst, c1)
    e2 = _rowdot(x, h2[batch])[:, 0]
    a2 = _seg_softmax(e2, batch, size)
    r2 = jax.ops.segment_sum(a2[:, None] * x, batch, num_segments=size)
    q_star = jnp.concatenate([h2, r2], axis=1)
    return jnp.where((cnt > 0)[:, None], q_star, 0.0)


def kernel(x_atom, atom_idx, x_group, group_idx, edge_index_group, cond_atom,
           g_proj_W, g_proj_b, fg1_W, fg1_b, fg2_W, fg2_b, fb1_W, fb1_b,
           fb2_W, fb2_b, a2g_W, a2g_b, s2sA_Wih, s2sA_Whh, s2sA_bih,
           s2sA_bhh, merge_W, merge_b, gcn_msg_W, gcn_msg_b, gcn_self_W,
           gcn_self_b, gcn_att, s2sG_Wih, s2sG_Whh, s2sG_bih, s2sG_bhh,
           g2a_W, g2a_b):
    Na = x_atom.shape[0]
    Gm = x_group.shape[0]
    Ninc = atom_idx.shape[0]

    xg_static = _mm(x_group[:, :40], g_proj_W, g_proj_b)
    xa_proj = _mm(x_atom, a2g_W, a2g_b)

    xa_items = xa_proj[atom_idx]
    cnt_g = jax.ops.segment_sum(jnp.ones((Ninc,), jnp.float32), group_idx,
                                num_segments=Gm)
    xg_a2g = _set2set(xa_items, group_idx, Gm,
                      s2sA_Wih, s2sA_Whh, s2sA_bih, s2sA_bhh, cnt_g)
    xg_from_atom = _mm(xg_a2g, merge_W, merge_b)

    cond_sum = jax.ops.segment_sum(cond_atom[atom_idx], group_idx,
                                   num_segments=Gm)
    cond_g = cond_sum / jnp.maximum(cnt_g, 1.0)[:, None]
    xg = _film(cond_g, fg1_W, fg1_b, fg2_W, fg2_b,
               fb1_W, fb1_b, fb2_W, fb2_b, xg_from_atom)

    src = edge_index_group[0]
    dst = edge_index_group[1]
    msgp = _mm(xg, gcn_msg_W, gcn_msg_b)
    att = gcn_att.reshape(1, -1)
    alpha_g = _mm(msgp, att, jnp.zeros((1,), jnp.float32))[:, 0]
    alpha = alpha_g[src]
    alpha = jnp.where(alpha >= 0, alpha, 0.2 * alpha)
    alpha = _seg_softmax(alpha, dst, Gm)
    agg = jax.ops.segment_sum(msgp[src] * alpha[:, None], dst,
                              num_segments=Gm)
    selfp = _mm(xg, gcn_self_W, gcn_self_b)
    xg2 = jnp.maximum(agg + selfp, 0.0)
    xg_out = jnp.concatenate([xg_static, xg2], axis=1)

    xg_items = xg_out[group_idx]
    cnt_a = jax.ops.segment_sum(jnp.ones((Ninc,), jnp.float32), atom_idx,
                                num_segments=Na)
    xa_g2a = _set2set(xg_items, atom_idx, Na,
                      s2sG_Wih, s2sG_Whh, s2sG_bih, s2sG_bhh, cnt_a)
    out_atom = _mm_add(xa_g2a, g2a_W, g2a_b, x_atom)
    return (out_atom, xg_out)


# speedup vs baseline: 1.3131x; 1.2228x over previous
import functools
import jax
import jax.numpy as jnp
from jax.experimental import pallas as pl


def _pick_bm(m):
    for c in (2000, 1000, 500, 250, 125, 100, 50, 25, 8, 5, 4, 2, 1):
        if m % c == 0:
            return c
    return 1


def _mm_body(x_ref, w_ref, b_ref, o_ref, *, act):
    y = jnp.dot(x_ref[...], w_ref[...], preferred_element_type=jnp.float32)
    y = y + b_ref[...]
    if act == "relu":
        y = jnp.maximum(y, 0.0)
    o_ref[...] = y


def _mm(x, W, b, act=None):
    M, K = x.shape
    N = W.shape[0]
    bm = _pick_bm(M)
    Wt = W.T
    b2 = jnp.asarray(b, jnp.float32).reshape(1, N)
    return pl.pallas_call(
        functools.partial(_mm_body, act=act),
        grid=(M // bm,),
        in_specs=[
            pl.BlockSpec((bm, K), lambda i: (i, 0)),
            pl.BlockSpec((K, N), lambda i: (0, 0)),
            pl.BlockSpec((1, N), lambda i: (0, 0)),
        ],
        out_specs=pl.BlockSpec((bm, N), lambda i: (i, 0)),
        out_shape=jax.ShapeDtypeStruct((M, N), jnp.float32),
    )(x, Wt, b2)


def _mm_add_body(x_ref, w_ref, b_ref, a_ref, o_ref):
    y = jnp.dot(x_ref[...], w_ref[...], preferred_element_type=jnp.float32)
    o_ref[...] = y + b_ref[...] + a_ref[...]


def _mm_add(x, W, b, add):
    M, K = x.shape
    N = W.shape[0]
    bm = _pick_bm(M)
    Wt = W.T
    b2 = jnp.asarray(b, jnp.float32).reshape(1, N)
    return pl.pallas_call(
        _mm_add_body,
        grid=(M // bm,),
        in_specs=[
            pl.BlockSpec((bm, K), lambda i: (i, 0)),
            pl.BlockSpec((K, N), lambda i: (0, 0)),
            pl.BlockSpec((1, N), lambda i: (0, 0)),
            pl.BlockSpec((bm, N), lambda i: (i, 0)),
        ],
        out_specs=pl.BlockSpec((bm, N), lambda i: (i, 0)),
        out_shape=jax.ShapeDtypeStruct((M, N), jnp.float32),
    )(x, Wt, b2, add)


def _lstm2_body(g_ref, gc_ref, c1_ref, h_ref, c_ref, *, d):
    g = g_ref[...] + gc_ref[...]
    i = g[:, 0:d]
    f = g[:, d:2 * d]
    gg = g[:, 2 * d:3 * d]
    o = g[:, 3 * d:4 * d]
    c = jax.nn.sigmoid(f) * c1_ref[...] + jax.nn.sigmoid(i) * jnp.tanh(gg)
    h = jax.nn.sigmoid(o) * jnp.tanh(c)
    h_ref[...] = h
    c_ref[...] = c


def _lstm_step2(r, Wih_r, gconst, c1):
    M, d = r.shape
    G = _mm(r, Wih_r, gconst)
    bm = _pick_bm(M)
    c1b = c1.reshape(1, d)
    zero4 = jnp.zeros((1, 4 * d), jnp.float32)
    h, c = pl.pallas_call(
        functools.partial(_lstm2_body, d=d),
        grid=(M // bm,),
        in_specs=[
            pl.BlockSpec((bm, 4 * d), lambda i: (i, 0)),
            pl.BlockSpec((1, 4 * d), lambda i: (0, 0)),
            pl.BlockSpec((1, d), lambda i: (0, 0)),
        ],
        out_specs=[
            pl.BlockSpec((bm, d), lambda i: (i, 0)),
            pl.BlockSpec((bm, d), lambda i: (i, 0)),
        ],
        out_shape=[
            jax.ShapeDtypeStruct((M, d), jnp.float32),
            jax.ShapeDtypeStruct((M, d), jnp.float32),
        ],
    )(G, zero4, c1b)
    return h, c


def _rowdot_body(x_ref, y_ref, o_ref):
    o_ref[...] = jnp.sum(x_ref[...] * y_ref[...], axis=1, keepdims=True)


def _rowdot(x, y):
    M, D = x.shape
    bm = _pick_bm(M)
    return pl.pallas_call(
        _rowdot_body,
        grid=(M // bm,),
        in_specs=[
            pl.BlockSpec((bm, D), lambda i: (i, 0)),
            pl.BlockSpec((bm, D), lambda i: (i, 0)),
        ],
        out_specs=pl.BlockSpec((bm, 1), lambda i: (i, 0)),
        out_shape=jax.ShapeDtypeStruct((M, 1), jnp.float32),
    )(x, y)


def _film_body(cg_ref, w1g_ref, b1g_ref, w2g_ref, b2g_ref,
               w1b_ref, b1b_ref, w2b_ref, b2b_ref, xfa_ref, o_ref):
    cg = cg_ref[...]
    hg = jnp.maximum(jnp.dot(cg, w1g_ref[...],
                             preferred_element_type=jnp.float32)
                     + b1g_ref[...], 0.0)
    gamma = jnp.dot(hg, w2g_ref[...],
                    preferred_element_type=jnp.float32) + b2g_ref[...]
    hb = jnp.maximum(jnp.dot(cg, w1b_ref[...],
                             preferred_element_type=jnp.float32)
                     + b1b_ref[...], 0.0)
    beta = jnp.dot(hb, w2b_ref[...],
                   preferred_element_type=jnp.float32) + b2b_ref[...]
    o_ref[...] = gamma * xfa_ref[...] + beta


def _film(cond_g, fg1_W, fg1_b, fg2_W, fg2_b, fb1_W, fb1_b, fb2_W, fb2_b,
          xg_from_atom):
    M, K = cond_g.shape
    H = fg1_W.shape[0]
    N = fg2_W.shape[0]
    bm = _pick_bm(M)
    args = (cond_g, fg1_W.T, fg1_b.reshape(1, H), fg2_W.T, fg2_b.reshape(1, N),
            fb1_W.T, fb1_b.reshape(1, H), fb2_W.T, fb2_b.reshape(1, N),
            xg_from_atom)
    return pl.pallas_call(
        _film_body,
        grid=(M // bm,),
        in_specs=[
            pl.BlockSpec((bm, K), lambda i: (i, 0)),
            pl.BlockSpec((K, H), lambda i: (0, 0)),
            pl.BlockSpec((1, H), lambda i: (0, 0)),
            pl.BlockSpec((H, N), lambda i: (0, 0)),
            pl.BlockSpec((1, N), lambda i: (0, 0)),
            pl.BlockSpec((K, H), lambda i: (0, 0)),
            pl.BlockSpec((1, H), lambda i: (0, 0)),
            pl.BlockSpec((H, N), lambda i: (0, 0)),
            pl.BlockSpec((1, N), lambda i: (0, 0)),
            pl.BlockSpec((bm, N), lambda i: (i, 0)),
        ],
        out_specs=pl.BlockSpec((bm, N), lambda i: (i, 0)),
        out_shape=jax.ShapeDtypeStruct((M, N), jnp.float32),
    )(*args)


def _lstm_const_step1(Wih, Whh, bih, bhh, d):
    g = bih + bhh
    i, f, gg, o = g[0:d], g[d:2 * d], g[2 * d:3 * d], g[3 * d:4 * d]
    c1 = jax.nn.sigmoid(i) * jnp.tanh(gg)
    h1 = jax.nn.sigmoid(o) * jnp.tanh(c1)
    return h1, c1


def _lstm_step2_consts(Wih, Whh, bih, bhh, h1, d):
    gconst = h1 @ Wih[:, 0:d].T + h1 @ Whh.T + bih + bhh
    Wih_r = Wih[:, d:2 * d]
    return Wih_r, gconst


def _seg_weighted_rowsum(e, seg, x, size):
    m = jax.ops.segment_max(e, seg, num_segments=size)
    m = jnp.where(jnp.isfinite(m), m, 0.0)
    ex = jnp.exp(e - m[seg])
    t = jax.ops.segment_sum(
        jnp.concatenate([ex[:, None] * x, ex[:, None]], axis=1),
        seg, num_segments=size)
    d = x.shape[1]
    return t[:, :d] / (t[:, d:d + 1] + 1e-16)


def _set2set(table, src_idx, x, batch, size, Wih, Whh, bih, bhh, cnt):
    d = x.shape[1]
    h1, c1 = _lstm_const_step1(Wih, Whh, bih, bhh, d)
    e1 = _mm(table, h1.reshape(1, d), jnp.zeros((1,), jnp.float32))[:, 0]
    r1 = _seg_weighted_rowsum(e1[src_idx], batch, x, size)
    Wih_r, gconst = _lstm_step2_consts(Wih, Whh, bih, bhh, h1, d)
    h2, c2 = _lstm_step2(r1, Wih_r, gconst, c1)
    e2 = _rowdot(x, h2[batch])[:, 0]
    r2 = _seg_weighted_rowsum(e2, batch, x, size)
    q_star = jnp.concatenate([h2, r2], axis=1)
    return jnp.where((cnt > 0)[:, None], q_star, 0.0)


def kernel(x_atom, atom_idx, x_group, group_idx, edge_index_group, cond_atom,
           g_proj_W, g_proj_b, fg1_W, fg1_b, fg2_W, fg2_b, fb1_W, fb1_b,
           fb2_W, fb2_b, a2g_W, a2g_b, s2sA_Wih, s2sA_Whh, s2sA_bih,
           s2sA_bhh, merge_W, merge_b, gcn_msg_W, gcn_msg_b, gcn_self_W,
           gcn_self_b, gcn_att, s2sG_Wih, s2sG_Whh, s2sG_bih, s2sG_bhh,
           g2a_W, g2a_b):
    Na = x_atom.shape[0]
    Gm = x_group.shape[0]
    Ninc = atom_idx.shape[0]

    xg_static = _mm(x_group[:, :40], g_proj_W, g_proj_b)
    xa_proj = _mm(x_atom, a2g_W, a2g_b)

    csel = jnp.concatenate(
        [cond_atom[atom_idx], jnp.ones((Ninc, 1), jnp.float32)], axis=1)
    ct = jax.ops.segment_sum(csel, group_idx, num_segments=Gm)
    cond_sum = ct[:, :-1]
    cnt_g = ct[:, -1]

    xa_items = xa_proj[atom_idx]
    xg_a2g = _set2set(xa_proj, atom_idx, xa_items, group_idx, Gm,
                      s2sA_Wih, s2sA_Whh, s2sA_bih, s2sA_bhh, cnt_g)
    xg_from_atom = _mm(xg_a2g, merge_W, merge_b)

    cond_g = cond_sum / jnp.maximum(cnt_g, 1.0)[:, None]
    xg = _film(cond_g, fg1_W, fg1_b, fg2_W, fg2_b,
               fb1_W, fb1_b, fb2_W, fb2_b, xg_from_atom)

    src = edge_index_group[0]
    dst = edge_index_group[1]
    msgp = _mm(xg, gcn_msg_W, gcn_msg_b)
    att = gcn_att.reshape(1, -1)
    alpha_g = _mm(msgp, att, jnp.zeros((1,), jnp.float32))[:, 0]
    alpha = alpha_g[src]
    alpha = jnp.where(alpha >= 0, alpha, 0.2 * alpha)
    agg = _seg_weighted_rowsum(alpha, dst, msgp[src], Gm)
    selfp = _mm(xg, gcn_self_W, gcn_self_b)
    xg2 = jnp.maximum(agg + selfp, 0.0)
    xg_out = jnp.concatenate([xg_static, xg2], axis=1)

    xg_items = xg_out[group_idx]
    cnt_a = jax.ops.segment_sum(jnp.ones((Ninc,), jnp.float32), atom_idx,
                                num_segments=Na)
    xa_g2a = _set2set(xg_out, group_idx, xg_items, atom_idx, Na,
                      s2sG_Wih, s2sG_Whh, s2sG_bih, s2sG_bhh, cnt_a)
    out_atom = _mm_add(xa_g2a, g2a_W, g2a_b, x_atom)
    return (out_atom, xg_out)
